# Initial kernel scaffold; baseline (speedup 1.0000x reference)
import jax
import jax.numpy as jnp
from jax.experimental import pallas as pl


def kernel(input_y, sigma):
    raise NotImplementedError("write your pallas kernel here")


# speedup vs baseline: 1.0362x; 1.0362x over previous
import jax
import jax.numpy as jnp
from jax.experimental import pallas as pl

_P1 = 7
_P2 = 7
_K1 = 18
_K2 = 55
_WIN = 37
_STEP = 4


def _reflect_pad(x, r):
    return jnp.pad(x, ((0, 0), (0, 0), (r, r), (r, r)), mode='reflect')


def _unfold(x, p):
    N, C, Hp, Wp = x.shape
    oh = Hp - p + 1
    ow = Wp - p + 1
    patches = jnp.stack(
        [x[:, :, i:i + oh, j:j + ow] for i in range(p) for j in range(p)],
        axis=2)
    return patches.reshape(N, C * p * p, oh * ow)


def _reflect_unfold(x, r, p):
    return _unfold(_reflect_pad(x, r), p)


def _fold(x, output_size, p):
    OH, OW = output_size
    oh = OH - p + 1
    ow = OW - p + 1
    N = x.shape[0]
    C = x.shape[1] // (p * p)
    xr = x.reshape(N, C, p, p, oh, ow)
    out = jnp.zeros((N, C, OH, OW), x.dtype)
    for i in range(p):
        for j in range(p):
            out = out.at[:, :, i:i + oh, j:j + ow].add(xr[:, :, i, j])
    return out


def _stack_neigh(x, p, inc):
    N, C, H, Wd = x.shape
    u = _reflect_unfold(x, p // 2 + inc, p)
    return u.reshape(N, p * p, H + 2 * inc, Wd + 2 * inc)


def _block_matching(input_x, m, p):
    N, C, H, Wd = input_x.shape
    w = _WIN
    s = _STEP
    x = _stack_neigh(input_x, p, w // 2)
    x_center = x[:, :, w // 2:H + w // 2:s, w // 2:Wd + w // 2:s]
    dists = []
    for i in range(w):
        for j in range(w):
            x_other = x[:, :, i:i + H:s, j:j + Wd:s]
            dists.append(jnp.sum((x_other - x_center) ** 2, axis=1))
    x_dist = jnp.stack(dists, axis=1)
    x_dist = x_dist.at[:, (w // 2) * w + w // 2].set(-1.0)
    _, idx = jax.lax.top_k(jnp.moveaxis(-x_dist, 1, -1), m)
    indices = jnp.moveaxis(idx, -1, 1)
    ind_rows = indices // w - w // 2
    ind_cols = indices % w - w // 2
    rows = jnp.arange(w // 2, H + w // 2, s)
    cols = jnp.arange(w // 2, Wd + w // 2, s)
    indices_row = ind_rows + rows[None, None, :, None]
    indices_col = ind_cols + cols[None, None, None, :]
    flat = indices_row * (Wd + w - 1) + indices_col
    Nb, mm, Hs, Ws = flat.shape
    flat = jnp.swapaxes(flat.reshape(Nb, mm, Hs * Ws), 1, 2).reshape(Nb, -1)
    return flat


def _group_patches(unfold_y, indices, m, n):
    N, nc, L = unfold_y.shape
    idx = jnp.broadcast_to(indices[:, None, :], (N, nc, indices.shape[1]))
    Y = jnp.take_along_axis(unfold_y, idx, axis=2)
    return jnp.swapaxes(Y, 1, 2).reshape(N, -1, m, n)


def _aggregation(Y, weights, indices, input_y, unfold_y, p):
    N, C, H, Wd = input_y.shape
    w = _WIN
    Yw = Y * weights
    Yp = jnp.transpose(Yw, (0, 3, 1, 2)).reshape(N, C * p * p, -1)
    wf = jnp.broadcast_to(weights.reshape(N, 1, -1), Yp.shape)
    bidx = jnp.arange(N)[:, None, None]
    cidx = jnp.arange(C * p * p)[None, :, None]
    col = indices[:, None, :]
    X_sum = jnp.zeros_like(unfold_y).at[bidx, cidx, col].add(Yp)
    divisor = jnp.zeros_like(unfold_y).at[bidx, cidx, col].add(wf)
    OH = H + w - 1 + p - 1
    OW = Wd + w - 1 + p - 1
    xx = _fold(X_sum, (OH, OW), p)
    dv = _fold(divisor, (OH, OW), p)
    den = xx / dv
    r = w // 2 + p // 2
    return den[:, :, r:-r, r:-r]


def _gram_kernel(c_ref, b_ref, g_ref, *, add_ridge):
    B = b_ref[0]
    G = jax.lax.dot_general(B, B, (((1,), (1,)), ((), ())),
                            preferred_element_type=jnp.float32)
    if add_ridge:
        m = G.shape[0]
        eye = (jax.lax.broadcasted_iota(jnp.int32, (m, m), 0) ==
               jax.lax.broadcasted_iota(jnp.int32, (m, m), 1))
        G = G + jnp.where(eye, c_ref[0, 0], 0.0)
    g_ref[0] = G


def _sweep_theta_kernel(c_ref, g_ref, theta_ref, w_ref, *, m):
    M = g_ref[...]
    rows_i = jax.lax.broadcasted_iota(jnp.int32, (1, m, m), 1)
    cols_i = jax.lax.broadcasted_iota(jnp.int32, (1, m, m), 2)
    shape = M.shape
    for j in range(m):
        p = M[:, j:j + 1, j:j + 1]
        col = M[:, :, j:j + 1]
        row = M[:, j:j + 1, :]
        inv_p = 1.0 / p
        M = M - col * row * inv_p
        M = jnp.where(rows_i == j, jnp.broadcast_to(row * inv_p, shape), M)
        M = jnp.where(cols_i == j, jnp.broadcast_to(col * inv_p, shape), M)
        M = jnp.where((rows_i == j) & (cols_i == j),
                      jnp.broadcast_to(-inv_p, shape), M)
    c = c_ref[0, 0]
    theta = jnp.where(rows_i == cols_i, 1.0, 0.0) + c * M
    theta_ref[...] = theta
    w_ref[...] = 1.0 / jnp.sum(theta * theta, axis=2)


def _xhat_kernel(t_ref, y_ref, o_ref):
    o_ref[0] = jax.lax.dot_general(t_ref[0], y_ref[0], (((1,), (0,)), ((), ())),
                                   preferred_element_type=jnp.float32)


def _denoise(Y, basis, sigma, add_ridge):
    _, Bn, m, n = Y.shape
    Yr = Y.reshape(Bn, m, n)
    Br = basis.reshape(Bn, m, n)
    c = (n * sigma ** 2).reshape(1, 1).astype(jnp.float32)

    gram = pl.pallas_call(
        lambda c_ref, b_ref, g_ref: _gram_kernel(c_ref, b_ref, g_ref,
                                                 add_ridge=add_ridge),
        grid=(Bn,),
        in_specs=[
            pl.BlockSpec((1, 1), lambda i: (0, 0)),
            pl.BlockSpec((1, m, n), lambda i: (i, 0, 0)),
        ],
        out_specs=pl.BlockSpec((1, m, m), lambda i: (i, 0, 0)),
        out_shape=jax.ShapeDtypeStruct((Bn, m, m), jnp.float32),
    )(c, Br)

    blk = 64
    nblk = Bn // blk
    theta, w = pl.pallas_call(
        lambda c_ref, g_ref, t_ref, w_ref: _sweep_theta_kernel(
            c_ref, g_ref, t_ref, w_ref, m=m),
        grid=(nblk,),
        in_specs=[
            pl.BlockSpec((1, 1), lambda i: (0, 0)),
            pl.BlockSpec((blk, m, m), lambda i: (i, 0, 0)),
        ],
        out_specs=[
            pl.BlockSpec((blk, m, m), lambda i: (i, 0, 0)),
            pl.BlockSpec((blk, m), lambda i: (i, 0)),
        ],
        out_shape=[
            jax.ShapeDtypeStruct((Bn, m, m), jnp.float32),
            jax.ShapeDtypeStruct((Bn, m), jnp.float32),
        ],
    )(c, gram)

    xhat = pl.pallas_call(
        _xhat_kernel,
        grid=(Bn,),
        in_specs=[
            pl.BlockSpec((1, m, m), lambda i: (i, 0, 0)),
            pl.BlockSpec((1, m, n), lambda i: (i, 0, 0)),
        ],
        out_specs=pl.BlockSpec((1, m, n), lambda i: (i, 0, 0)),
        out_shape=jax.ShapeDtypeStruct((Bn, m, n), jnp.float32),
    )(theta, Yr)

    return xhat.reshape(1, Bn, m, n), w.reshape(1, Bn, m, 1)


def _step1(input_y, sigma):
    N, C, H, Wd = input_y.shape
    p, m, w = _P1, _K1, _WIN
    y_block = jnp.mean(input_y, axis=1, keepdims=True)
    indices = _block_matching(y_block, m, p)
    unfold_y = _reflect_unfold(input_y, p // 2 + w // 2, p)
    Y = _group_patches(unfold_y, indices, m, C * p * p)
    X_hat, weights = _denoise(Y, Y, sigma, add_ridge=False)
    return _aggregation(X_hat, weights, indices, input_y, unfold_y, p)


def _step2(input_y, input_x, sigma):
    N, C, H, Wd = input_y.shape
    p, m, w = _P2, _K2, _WIN
    x_block = jnp.mean(input_x, axis=1, keepdims=True)
    indices = _block_matching(x_block, m, p)
    unfold_y = _reflect_unfold(input_y, p // 2 + w // 2, p)
    unfold_x = _reflect_unfold(input_x, p // 2 + w // 2, p)
    Y = _group_patches(unfold_y, indices, m, C * p * p)
    X = _group_patches(unfold_x, indices, m, C * p * p)
    X_hat, weights = _denoise(Y, X, sigma, add_ridge=True)
    return _aggregation(X_hat, weights, indices, input_y, unfold_y, p)


def kernel(input_y, sigma):
    den1 = _step1(input_y, sigma)
    den2 = _step2(input_y, den1, sigma)
    return den2


# speedup vs baseline: 1.3425x; 1.2957x over previous
import jax
import jax.numpy as jnp
from jax.experimental import pallas as pl
from jax.experimental.pallas import tpu as pltpu

_P1 = 7
_P2 = 7
_K1 = 18
_K2 = 55
_WIN = 37
_STEP = 4


def _reflect_pad(x, r):
    return jnp.pad(x, ((0, 0), (0, 0), (r, r), (r, r)), mode='reflect')


def _unfold(x, p):
    N, C, Hp, Wp = x.shape
    oh = Hp - p + 1
    ow = Wp - p + 1
    patches = jnp.stack(
        [x[:, :, i:i + oh, j:j + ow] for i in range(p) for j in range(p)],
        axis=2)
    return patches.reshape(N, C * p * p, oh * ow)


def _reflect_unfold(x, r, p):
    return _unfold(_reflect_pad(x, r), p)


def _fold(x, output_size, p):
    OH, OW = output_size
    oh = OH - p + 1
    ow = OW - p + 1
    N = x.shape[0]
    C = x.shape[1] // (p * p)
    xr = x.reshape(N, C, p, p, oh, ow)
    out = jnp.zeros((N, C, OH, OW), x.dtype)
    for i in range(p):
        for j in range(p):
            out = out.at[:, :, i:i + oh, j:j + ow].add(xr[:, :, i, j])
    return out


def _stack_neigh(x, p, inc):
    N, C, H, Wd = x.shape
    u = _reflect_unfold(x, p // 2 + inc, p)
    return u.reshape(N, p * p, H + 2 * inc, Wd + 2 * inc)


def _dist_topk_kernel(ims_ref, c_ref, o_ref, *, m):
    qr = pl.program_id(0)
    acc = jnp.zeros((37, 37, 56), jnp.float32)
    for di in range(7):
        for dj in range(7):
            k = di * 7 + dj
            blk = ims_ref[dj, pl.ds(di + 4 * qr, 37), :, :]
            cv = c_ref[0, k, :]
            d = blk - cv[None, None, :]
            acc = acc + d * d
    ii = jax.lax.broadcasted_iota(jnp.int32, (37, 37, 56), 0)
    jj = jax.lax.broadcasted_iota(jnp.int32, (37, 37, 56), 1)
    acc = jnp.where((ii == 18) & (jj == 18), -1.0, acc)
    of = ii * 37 + jj
    big = jnp.float32(3.0e38)
    for r in range(m):
        mn = jnp.min(acc, axis=(0, 1))
        hit = acc == mn[None, None, :]
        idxv = jnp.min(jnp.where(hit, of, 1369), axis=(0, 1))
        o_ref[0, r, :] = idxv
        acc = jnp.where(of == idxv[None, None, :], big, acc)


def _block_matching_pl(xb, m):
    xpad = jnp.pad(xb, ((0, 0), (0, 0), (21, 21), (21, 21)),
                   mode='reflect')[0, 0]
    A = jnp.stack([xpad[:, t:t + 221:4] for t in range(43)], axis=1)
    IMs = jnp.stack([A[:, dj:dj + 37, :] for dj in range(7)], axis=0)
    C2 = jnp.stack(
        [xpad[18 + di:18 + di + 221:4, 18 + dj:18 + dj + 221:4]
         for di in range(7) for dj in range(7)], axis=1)
    o = pl.pallas_call(
        lambda a, b, c: _dist_topk_kernel(a, b, c, m=m),
        grid=(56,),
        in_specs=[
            pl.BlockSpec((7, 266, 37, 56), lambda i: (0, 0, 0, 0)),
            pl.BlockSpec((1, 49, 56), lambda i: (i, 0, 0)),
        ],
        out_specs=pl.BlockSpec((1, m, 56), lambda i: (i, 0, 0)),
        out_shape=jax.ShapeDtypeStruct((56, m, 56), jnp.int32),
    )(IMs, C2)
    qr = jnp.arange(56)[:, None, None]
    qc = jnp.arange(56)[None, None, :]
    flat = (o // 37 + 4 * qr) * 260 + (o % 37 + 4 * qc)
    return jnp.transpose(flat, (0, 2, 1)).reshape(3136, m)


def _scatter_kernel(p_ref, i_ref, o_ref, *, m, base):
    g = pl.program_id(0)

    @pl.when(g == 0)
    def _():
        o_ref[...] = jnp.zeros_like(o_ref)

    def body(k, carry):
        pos = i_ref[0, 0, k] - base
        row = p_ref[0, pl.ds(k, 1), :]
        o_ref[pl.ds(pos, 1), :] = o_ref[pl.ds(pos, 1), :] + row
        return carry

    jax.lax.fori_loop(0, m, body, 0)


def _aggregation_pl(X_hat, weights, idx2, input_y, p):
    N, C, H, Wd = input_y.shape
    w = _WIN
    Bn, m = idx2.shape
    L = (H + w - 1) * (Wd + w - 1)
    P = jnp.concatenate([X_hat * weights, weights], axis=3).reshape(Bn, m, 148)
    idx3 = idx2.reshape(Bn, 1, m)
    half = Bn // 2
    hrows = 145 * 260
    bases = (0, 112 * 260)
    halves = []
    for h in range(2):
        halves.append(pl.pallas_call(
            lambda a, b, c, _h=h: _scatter_kernel(a, b, c, m=m,
                                                  base=bases[_h]),
            grid=(half,),
            in_specs=[
                pl.BlockSpec((1, m, 148), lambda i, _h=h: (_h * half + i, 0, 0)),
                pl.BlockSpec((1, 1, m), lambda i, _h=h: (_h * half + i, 0, 0),
                             memory_space=pltpu.SMEM),
            ],
            out_specs=pl.BlockSpec((hrows, 148), lambda i: (0, 0)),
            out_shape=jax.ShapeDtypeStruct((hrows, 148), jnp.float32),
        )(P, idx3))
    acc = (jnp.zeros((L, 148), jnp.float32)
           .at[:hrows].add(halves[0])
           .at[bases[1]:bases[1] + hrows].add(halves[1]))
    X_sum = acc[:, :147].T.reshape(1, 147, L)
    dsum = acc[:, 147]
    OH = H + w - 1 + p - 1
    OW = Wd + w - 1 + p - 1
    xx = _fold(X_sum, (OH, OW), p)
    dv = _fold(jnp.broadcast_to(dsum[None, None, :], (1, p * p, L)),
               (OH, OW), p)
    den = xx / dv
    r = w // 2 + p // 2
    return den[:, :, r:-r, r:-r]


def _block_matching(input_x, m, p):
    N, C, H, Wd = input_x.shape
    w = _WIN
    s = _STEP
    x = _stack_neigh(input_x, p, w // 2)
    x_center = x[:, :, w // 2:H + w // 2:s, w // 2:Wd + w // 2:s]
    dists = []
    for i in range(w):
        for j in range(w):
            x_other = x[:, :, i:i + H:s, j:j + Wd:s]
            dists.append(jnp.sum((x_other - x_center) ** 2, axis=1))
    x_dist = jnp.stack(dists, axis=1)
    x_dist = x_dist.at[:, (w // 2) * w + w // 2].set(-1.0)
    _, idx = jax.lax.top_k(jnp.moveaxis(-x_dist, 1, -1), m)
    indices = jnp.moveaxis(idx, -1, 1)
    ind_rows = indices // w - w // 2
    ind_cols = indices % w - w // 2
    rows = jnp.arange(w // 2, H + w // 2, s)
    cols = jnp.arange(w // 2, Wd + w // 2, s)
    indices_row = ind_rows + rows[None, None, :, None]
    indices_col = ind_cols + cols[None, None, None, :]
    flat = indices_row * (Wd + w - 1) + indices_col
    Nb, mm, Hs, Ws = flat.shape
    flat = jnp.swapaxes(flat.reshape(Nb, mm, Hs * Ws), 1, 2).reshape(Nb, -1)
    return flat


def _group_patches(unfold_y, indices, m, n):
    N, nc, L = unfold_y.shape
    idx = jnp.broadcast_to(indices[:, None, :], (N, nc, indices.shape[1]))
    Y = jnp.take_along_axis(unfold_y, idx, axis=2)
    return jnp.swapaxes(Y, 1, 2).reshape(N, -1, m, n)


def _aggregation(Y, weights, indices, input_y, unfold_y, p):
    N, C, H, Wd = input_y.shape
    w = _WIN
    Yw = Y * weights
    Yp = jnp.transpose(Yw, (0, 3, 1, 2)).reshape(N, C * p * p, -1)
    wf = jnp.broadcast_to(weights.reshape(N, 1, -1), Yp.shape)
    bidx = jnp.arange(N)[:, None, None]
    cidx = jnp.arange(C * p * p)[None, :, None]
    col = indices[:, None, :]
    X_sum = jnp.zeros_like(unfold_y).at[bidx, cidx, col].add(Yp)
    divisor = jnp.zeros_like(unfold_y).at[bidx, cidx, col].add(wf)
    OH = H + w - 1 + p - 1
    OW = Wd + w - 1 + p - 1
    xx = _fold(X_sum, (OH, OW), p)
    dv = _fold(divisor, (OH, OW), p)
    den = xx / dv
    r = w // 2 + p // 2
    return den[:, :, r:-r, r:-r]


def _gram_kernel(c_ref, b_ref, g_ref, *, add_ridge):
    B = b_ref[0]
    G = jax.lax.dot_general(B, B, (((1,), (1,)), ((), ())),
                            preferred_element_type=jnp.float32)
    if add_ridge:
        m = G.shape[0]
        eye = (jax.lax.broadcasted_iota(jnp.int32, (m, m), 0) ==
               jax.lax.broadcasted_iota(jnp.int32, (m, m), 1))
        G = G + jnp.where(eye, c_ref[0, 0], 0.0)
    g_ref[0] = G


def _sweep_theta_kernel(c_ref, g_ref, theta_ref, w_ref, *, m):
    M = g_ref[...]
    rows_i = jax.lax.broadcasted_iota(jnp.int32, (1, m, m), 1)
    cols_i = jax.lax.broadcasted_iota(jnp.int32, (1, m, m), 2)
    shape = M.shape
    for j in range(m):
        p = M[:, j:j + 1, j:j + 1]
        col = M[:, :, j:j + 1]
        row = M[:, j:j + 1, :]
        inv_p = 1.0 / p
        M = M - col * row * inv_p
        M = jnp.where(rows_i == j, jnp.broadcast_to(row * inv_p, shape), M)
        M = jnp.where(cols_i == j, jnp.broadcast_to(col * inv_p, shape), M)
        M = jnp.where((rows_i == j) & (cols_i == j),
                      jnp.broadcast_to(-inv_p, shape), M)
    c = c_ref[0, 0]
    theta = jnp.where(rows_i == cols_i, 1.0, 0.0) + c * M
    theta_ref[...] = theta
    w_ref[...] = 1.0 / jnp.sum(theta * theta, axis=2)


def _xhat_kernel(t_ref, y_ref, o_ref):
    o_ref[0] = jax.lax.dot_general(t_ref[0], y_ref[0], (((1,), (0,)), ((), ())),
                                   preferred_element_type=jnp.float32)


def _denoise(Y, basis, sigma, add_ridge):
    _, Bn, m, n = Y.shape
    Yr = Y.reshape(Bn, m, n)
    Br = basis.reshape(Bn, m, n)
    c = (n * sigma ** 2).reshape(1, 1).astype(jnp.float32)

    gram = pl.pallas_call(
        lambda c_ref, b_ref, g_ref: _gram_kernel(c_ref, b_ref, g_ref,
                                                 add_ridge=add_ridge),
        grid=(Bn,),
        in_specs=[
            pl.BlockSpec((1, 1), lambda i: (0, 0)),
            pl.BlockSpec((1, m, n), lambda i: (i, 0, 0)),
        ],
        out_specs=pl.BlockSpec((1, m, m), lambda i: (i, 0, 0)),
        out_shape=jax.ShapeDtypeStruct((Bn, m, m), jnp.float32),
    )(c, Br)

    blk = 64
    nblk = Bn // blk
    theta, w = pl.pallas_call(
        lambda c_ref, g_ref, t_ref, w_ref: _sweep_theta_kernel(
            c_ref, g_ref, t_ref, w_ref, m=m),
        grid=(nblk,),
        in_specs=[
            pl.BlockSpec((1, 1), lambda i: (0, 0)),
            pl.BlockSpec((blk, m, m), lambda i: (i, 0, 0)),
        ],
        out_specs=[
            pl.BlockSpec((blk, m, m), lambda i: (i, 0, 0)),
            pl.BlockSpec((blk, m), lambda i: (i, 0)),
        ],
        out_shape=[
            jax.ShapeDtypeStruct((Bn, m, m), jnp.float32),
            jax.ShapeDtypeStruct((Bn, m), jnp.float32),
        ],
    )(c, gram)

    xhat = pl.pallas_call(
        _xhat_kernel,
        grid=(Bn,),
        in_specs=[
            pl.BlockSpec((1, m, m), lambda i: (i, 0, 0)),
            pl.BlockSpec((1, m, n), lambda i: (i, 0, 0)),
        ],
        out_specs=pl.BlockSpec((1, m, n), lambda i: (i, 0, 0)),
        out_shape=jax.ShapeDtypeStruct((Bn, m, n), jnp.float32),
    )(theta, Yr)

    return xhat.reshape(1, Bn, m, n), w.reshape(1, Bn, m, 1)


def _step1(input_y, sigma):
    N, C, H, Wd = input_y.shape
    p, m, w = _P1, _K1, _WIN
    y_block = jnp.mean(input_y, axis=1, keepdims=True)
    idx2 = _block_matching_pl(y_block, m)
    indices = idx2.reshape(1, -1)
    unfold_y = _reflect_unfold(input_y, p // 2 + w // 2, p)
    Y = _group_patches(unfold_y, indices, m, C * p * p)
    X_hat, weights = _denoise(Y, Y, sigma, add_ridge=False)
    return _aggregation_pl(X_hat, weights, idx2, input_y, p)


def _step2(input_y, input_x, sigma):
    N, C, H, Wd = input_y.shape
    p, m, w = _P2, _K2, _WIN
    x_block = jnp.mean(input_x, axis=1, keepdims=True)
    idx2 = _block_matching_pl(x_block, m)
    indices = idx2.reshape(1, -1)
    unfold_y = _reflect_unfold(input_y, p // 2 + w // 2, p)
    unfold_x = _reflect_unfold(input_x, p // 2 + w // 2, p)
    Y = _group_patches(unfold_y, indices, m, C * p * p)
    X = _group_patches(unfold_x, indices, m, C * p * p)
    X_hat, weights = _denoise(Y, X, sigma, add_ridge=True)
    return _aggregation_pl(X_hat, weights, idx2, input_y, p)


def kernel(input_y, sigma):
    den1 = _step1(input_y, sigma)
    den2 = _step2(input_y, den1, sigma)
    return den2


# speedup vs baseline: 39.6505x; 29.5340x over previous
import jax
import jax.numpy as jnp
from jax.experimental import pallas as pl
from jax.experimental.pallas import tpu as pltpu

_P1 = 7
_P2 = 7
_K1 = 18
_K2 = 55
_WIN = 37
_STEP = 4


def _reflect_pad(x, r):
    return jnp.pad(x, ((0, 0), (0, 0), (r, r), (r, r)), mode='reflect')


def _unfold(x, p):
    N, C, Hp, Wp = x.shape
    oh = Hp - p + 1
    ow = Wp - p + 1
    patches = jnp.stack(
        [x[:, :, i:i + oh, j:j + ow] for i in range(p) for j in range(p)],
        axis=2)
    return patches.reshape(N, C * p * p, oh * ow)


def _reflect_unfold(x, r, p):
    return _unfold(_reflect_pad(x, r), p)


def _fold(x, output_size, p):
    OH, OW = output_size
    oh = OH - p + 1
    ow = OW - p + 1
    N = x.shape[0]
    C = x.shape[1] // (p * p)
    xr = x.reshape(N, C, p, p, oh, ow)
    out = jnp.zeros((N, C, OH, OW), x.dtype)
    for i in range(p):
        for j in range(p):
            out = out.at[:, :, i:i + oh, j:j + ow].add(xr[:, :, i, j])
    return out


def _stack_neigh(x, p, inc):
    N, C, H, Wd = x.shape
    u = _reflect_unfold(x, p // 2 + inc, p)
    return u.reshape(N, p * p, H + 2 * inc, Wd + 2 * inc)


def _dist_topk_kernel(a_ref, c_ref, o_ref, *, m):
    qr = pl.program_id(0)
    acc = jnp.zeros((37, 37, 56), jnp.float32)
    for di in range(7):
        for dj in range(7):
            k = di * 7 + dj
            blk = a_ref[pl.ds(di + 4 * qr, 37), dj:dj + 37, :]
            cv = c_ref[0, k, :]
            d = blk - cv[None, None, :]
            acc = acc + d * d
    ii = jax.lax.broadcasted_iota(jnp.int32, (37, 37, 56), 0)
    jj = jax.lax.broadcasted_iota(jnp.int32, (37, 37, 56), 1)
    acc = jnp.where((ii == 18) & (jj == 18), -1.0, acc)
    of = ii * 37 + jj
    big = jnp.float32(3.0e38)
    for r in range(m):
        mn = jnp.min(acc, axis=(0, 1))
        hit = acc == mn[None, None, :]
        idxv = jnp.min(jnp.where(hit, of, 1369), axis=(0, 1))
        o_ref[0, r, :] = idxv
        acc = jnp.where(of == idxv[None, None, :], big, acc)


def _block_matching_pl(xb, m):
    xpad = jnp.pad(xb, ((0, 0), (0, 0), (21, 21), (21, 21)),
                   mode='reflect')[0, 0]
    A = jnp.stack([xpad[:, t:t + 221:4] for t in range(43)], axis=1)
    C2 = jnp.stack(
        [xpad[18 + di:18 + di + 221:4, 18 + dj:18 + dj + 221:4]
         for di in range(7) for dj in range(7)], axis=1)
    o = pl.pallas_call(
        lambda a, b, c: _dist_topk_kernel(a, b, c, m=m),
        grid=(56,),
        in_specs=[
            pl.BlockSpec((266, 43, 56), lambda i: (0, 0, 0)),
            pl.BlockSpec((1, 49, 56), lambda i: (i, 0, 0)),
        ],
        out_specs=pl.BlockSpec((1, m, 56), lambda i: (i, 0, 0)),
        out_shape=jax.ShapeDtypeStruct((56, m, 56), jnp.int32),
    )(A, C2)
    qr = jnp.arange(56)[:, None, None]
    qc = jnp.arange(56)[None, None, :]
    flat = (o // 37 + 4 * qr) * 260 + (o % 37 + 4 * qc)
    return jnp.transpose(flat, (0, 2, 1)).reshape(3136, m)


def _gather_kernel(t_ref, i_ref, o_ref, *, m, base):
    def body(k, carry):
        pos = i_ref[0, 0, k] - base
        o_ref[0, pl.ds(k, 1), :] = t_ref[pl.ds(pos, 1), :]
        return carry

    jax.lax.fori_loop(0, m, body, 0)


def _group_patches_pl(table_t, idx2):
    Bn, m = idx2.shape
    half = Bn // 2
    hrows = 145 * 260
    bases = (0, 112 * 260)
    idx3 = idx2.reshape(Bn, 1, m)
    outs = []
    for h in range(2):
        table_h = jax.lax.dynamic_slice(table_t, (bases[h], 0), (hrows, 147))
        outs.append(pl.pallas_call(
            lambda a, b, c, _h=h: _gather_kernel(a, b, c, m=m,
                                                 base=bases[_h]),
            grid=(half,),
            in_specs=[
                pl.BlockSpec((hrows, 147), lambda i: (0, 0)),
                pl.BlockSpec((1, 1, m), lambda i, _h=h: (_h * half + i, 0, 0),
                             memory_space=pltpu.SMEM),
            ],
            out_specs=pl.BlockSpec((1, m, 147), lambda i: (i, 0, 0)),
            out_shape=jax.ShapeDtypeStruct((half, m, 147), jnp.float32),
        )(table_h, idx3))
    return jnp.concatenate(outs, axis=0).reshape(1, Bn, m, 147)


def _scatter_kernel(p_ref, i_ref, o_ref, *, m, base):
    g = pl.program_id(0)

    @pl.when(g == 0)
    def _():
        o_ref[...] = jnp.zeros_like(o_ref)

    def body(k, carry):
        pos = i_ref[0, 0, k] - base
        row = p_ref[0, pl.ds(k, 1), :]
        o_ref[pl.ds(pos, 1), :] = o_ref[pl.ds(pos, 1), :] + row
        return carry

    jax.lax.fori_loop(0, m, body, 0)


def _aggregation_pl(X_hat, weights, idx2, input_y, p):
    N, C, H, Wd = input_y.shape
    w = _WIN
    Bn, m = idx2.shape
    L = (H + w - 1) * (Wd + w - 1)
    P = jnp.concatenate([X_hat * weights, weights], axis=3).reshape(Bn, m, 148)
    idx3 = idx2.reshape(Bn, 1, m)
    half = Bn // 2
    hrows = 145 * 260
    bases = (0, 112 * 260)
    halves = []
    for h in range(2):
        halves.append(pl.pallas_call(
            lambda a, b, c, _h=h: _scatter_kernel(a, b, c, m=m,
                                                  base=bases[_h]),
            grid=(half,),
            in_specs=[
                pl.BlockSpec((1, m, 148), lambda i, _h=h: (_h * half + i, 0, 0)),
                pl.BlockSpec((1, 1, m), lambda i, _h=h: (_h * half + i, 0, 0),
                             memory_space=pltpu.SMEM),
            ],
            out_specs=pl.BlockSpec((hrows, 148), lambda i: (0, 0)),
            out_shape=jax.ShapeDtypeStruct((hrows, 148), jnp.float32),
        )(P, idx3))
    acc = (jnp.zeros((L, 148), jnp.float32)
           .at[:hrows].add(halves[0])
           .at[bases[1]:bases[1] + hrows].add(halves[1]))
    X_sum = acc[:, :147].T.reshape(1, 147, L)
    dsum = acc[:, 147]
    OH = H + w - 1 + p - 1
    OW = Wd + w - 1 + p - 1
    xx = _fold(X_sum, (OH, OW), p)
    dv = _fold(jnp.broadcast_to(dsum[None, None, :], (1, p * p, L)),
               (OH, OW), p)
    den = xx / dv
    r = w // 2 + p // 2
    return den[:, :, r:-r, r:-r]


def _block_matching(input_x, m, p):
    N, C, H, Wd = input_x.shape
    w = _WIN
    s = _STEP
    x = _stack_neigh(input_x, p, w // 2)
    x_center = x[:, :, w // 2:H + w // 2:s, w // 2:Wd + w // 2:s]
    dists = []
    for i in range(w):
        for j in range(w):
            x_other = x[:, :, i:i + H:s, j:j + Wd:s]
            dists.append(jnp.sum((x_other - x_center) ** 2, axis=1))
    x_dist = jnp.stack(dists, axis=1)
    x_dist = x_dist.at[:, (w // 2) * w + w // 2].set(-1.0)
    _, idx = jax.lax.top_k(jnp.moveaxis(-x_dist, 1, -1), m)
    indices = jnp.moveaxis(idx, -1, 1)
    ind_rows = indices // w - w // 2
    ind_cols = indices % w - w // 2
    rows = jnp.arange(w // 2, H + w // 2, s)
    cols = jnp.arange(w // 2, Wd + w // 2, s)
    indices_row = ind_rows + rows[None, None, :, None]
    indices_col = ind_cols + cols[None, None, None, :]
    flat = indices_row * (Wd + w - 1) + indices_col
    Nb, mm, Hs, Ws = flat.shape
    flat = jnp.swapaxes(flat.reshape(Nb, mm, Hs * Ws), 1, 2).reshape(Nb, -1)
    return flat


def _group_patches(unfold_y, indices, m, n):
    N, nc, L = unfold_y.shape
    idx = jnp.broadcast_to(indices[:, None, :], (N, nc, indices.shape[1]))
    Y = jnp.take_along_axis(unfold_y, idx, axis=2)
    return jnp.swapaxes(Y, 1, 2).reshape(N, -1, m, n)


def _aggregation(Y, weights, indices, input_y, unfold_y, p):
    N, C, H, Wd = input_y.shape
    w = _WIN
    Yw = Y * weights
    Yp = jnp.transpose(Yw, (0, 3, 1, 2)).reshape(N, C * p * p, -1)
    wf = jnp.broadcast_to(weights.reshape(N, 1, -1), Yp.shape)
    bidx = jnp.arange(N)[:, None, None]
    cidx = jnp.arange(C * p * p)[None, :, None]
    col = indices[:, None, :]
    X_sum = jnp.zeros_like(unfold_y).at[bidx, cidx, col].add(Yp)
    divisor = jnp.zeros_like(unfold_y).at[bidx, cidx, col].add(wf)
    OH = H + w - 1 + p - 1
    OW = Wd + w - 1 + p - 1
    xx = _fold(X_sum, (OH, OW), p)
    dv = _fold(divisor, (OH, OW), p)
    den = xx / dv
    r = w // 2 + p // 2
    return den[:, :, r:-r, r:-r]


def _gram_kernel(c_ref, b_ref, g_ref, *, add_ridge):
    B = b_ref[0]
    G = jax.lax.dot_general(B, B, (((1,), (1,)), ((), ())),
                            preferred_element_type=jnp.float32)
    if add_ridge:
        m = G.shape[0]
        eye = (jax.lax.broadcasted_iota(jnp.int32, (m, m), 0) ==
               jax.lax.broadcasted_iota(jnp.int32, (m, m), 1))
        G = G + jnp.where(eye, c_ref[0, 0], 0.0)
    g_ref[0] = G


def _sweep_theta_kernel(c_ref, g_ref, theta_ref, w_ref, *, m):
    M = g_ref[...]
    rows_i = jax.lax.broadcasted_iota(jnp.int32, (1, m, m), 1)
    cols_i = jax.lax.broadcasted_iota(jnp.int32, (1, m, m), 2)
    shape = M.shape
    for j in range(m):
        p = M[:, j:j + 1, j:j + 1]
        col = M[:, :, j:j + 1]
        row = M[:, j:j + 1, :]
        inv_p = 1.0 / p
        M = M - col * row * inv_p
        M = jnp.where(rows_i == j, jnp.broadcast_to(row * inv_p, shape), M)
        M = jnp.where(cols_i == j, jnp.broadcast_to(col * inv_p, shape), M)
        M = jnp.where((rows_i == j) & (cols_i == j),
                      jnp.broadcast_to(-inv_p, shape), M)
    c = c_ref[0, 0]
    theta = jnp.where(rows_i == cols_i, 1.0, 0.0) + c * M
    theta_ref[...] = theta
    w_ref[...] = 1.0 / jnp.sum(theta * theta, axis=2)


def _xhat_kernel(t_ref, y_ref, o_ref):
    o_ref[0] = jax.lax.dot_general(t_ref[0], y_ref[0], (((1,), (0,)), ((), ())),
                                   preferred_element_type=jnp.float32)


def _denoise(Y, basis, sigma, add_ridge):
    _, Bn, m, n = Y.shape
    Yr = Y.reshape(Bn, m, n)
    Br = basis.reshape(Bn, m, n)
    c = (n * sigma ** 2).reshape(1, 1).astype(jnp.float32)

    gram = pl.pallas_call(
        lambda c_ref, b_ref, g_ref: _gram_kernel(c_ref, b_ref, g_ref,
                                                 add_ridge=add_ridge),
        grid=(Bn,),
        in_specs=[
            pl.BlockSpec((1, 1), lambda i: (0, 0)),
            pl.BlockSpec((1, m, n), lambda i: (i, 0, 0)),
        ],
        out_specs=pl.BlockSpec((1, m, m), lambda i: (i, 0, 0)),
        out_shape=jax.ShapeDtypeStruct((Bn, m, m), jnp.float32),
    )(c, Br)

    blk = 64
    nblk = Bn // blk
    theta, w = pl.pallas_call(
        lambda c_ref, g_ref, t_ref, w_ref: _sweep_theta_kernel(
            c_ref, g_ref, t_ref, w_ref, m=m),
        grid=(nblk,),
        in_specs=[
            pl.BlockSpec((1, 1), lambda i: (0, 0)),
            pl.BlockSpec((blk, m, m), lambda i: (i, 0, 0)),
        ],
        out_specs=[
            pl.BlockSpec((blk, m, m), lambda i: (i, 0, 0)),
            pl.BlockSpec((blk, m), lambda i: (i, 0)),
        ],
        out_shape=[
            jax.ShapeDtypeStruct((Bn, m, m), jnp.float32),
            jax.ShapeDtypeStruct((Bn, m), jnp.float32),
        ],
    )(c, gram)

    xhat = pl.pallas_call(
        _xhat_kernel,
        grid=(Bn,),
        in_specs=[
            pl.BlockSpec((1, m, m), lambda i: (i, 0, 0)),
            pl.BlockSpec((1, m, n), lambda i: (i, 0, 0)),
        ],
        out_specs=pl.BlockSpec((1, m, n), lambda i: (i, 0, 0)),
        out_shape=jax.ShapeDtypeStruct((Bn, m, n), jnp.float32),
    )(theta, Yr)

    return xhat.reshape(1, Bn, m, n), w.reshape(1, Bn, m, 1)


def _step1(input_y, sigma):
    N, C, H, Wd = input_y.shape
    p, m, w = _P1, _K1, _WIN
    y_block = jnp.mean(input_y, axis=1, keepdims=True)
    idx2 = _block_matching_pl(y_block, m)
    unfold_y = _reflect_unfold(input_y, p // 2 + w // 2, p)
    Y = _group_patches_pl(unfold_y[0].T, idx2)
    X_hat, weights = _denoise(Y, Y, sigma, add_ridge=False)
    return _aggregation_pl(X_hat, weights, idx2, input_y, p)


def _step2(input_y, input_x, sigma):
    N, C, H, Wd = input_y.shape
    p, m, w = _P2, _K2, _WIN
    x_block = jnp.mean(input_x, axis=1, keepdims=True)
    idx2 = _block_matching_pl(x_block, m)
    unfold_y = _reflect_unfold(input_y, p // 2 + w // 2, p)
    unfold_x = _reflect_unfold(input_x, p // 2 + w // 2, p)
    Y = _group_patches_pl(unfold_y[0].T, idx2)
    X = _group_patches_pl(unfold_x[0].T, idx2)
    X_hat, weights = _denoise(Y, X, sigma, add_ridge=True)
    return _aggregation_pl(X_hat, weights, idx2, input_y, p)


def kernel(input_y, sigma):
    den1 = _step1(input_y, sigma)
    den2 = _step2(input_y, den1, sigma)
    return den2


# speedup vs baseline: 39.6954x; 1.0011x over previous
import jax
import jax.numpy as jnp
from jax.experimental import pallas as pl
from jax.experimental.pallas import tpu as pltpu

_P1 = 7
_P2 = 7
_K1 = 18
_K2 = 55
_WIN = 37
_STEP = 4


def _reflect_pad(x, r):
    return jnp.pad(x, ((0, 0), (0, 0), (r, r), (r, r)), mode='reflect')


def _unfold(x, p):
    N, C, Hp, Wp = x.shape
    oh = Hp - p + 1
    ow = Wp - p + 1
    patches = jnp.stack(
        [x[:, :, i:i + oh, j:j + ow] for i in range(p) for j in range(p)],
        axis=2)
    return patches.reshape(N, C * p * p, oh * ow)


def _reflect_unfold(x, r, p):
    return _unfold(_reflect_pad(x, r), p)


def _fold(x, output_size, p):
    OH, OW = output_size
    oh = OH - p + 1
    ow = OW - p + 1
    N = x.shape[0]
    C = x.shape[1] // (p * p)
    xr = x.reshape(N, C, p, p, oh, ow)
    out = jnp.zeros((N, C, OH, OW), x.dtype)
    for i in range(p):
        for j in range(p):
            out = out.at[:, :, i:i + oh, j:j + ow].add(xr[:, :, i, j])
    return out


def _dist_topk_kernel(a_ref, c_ref, o_ref, *, m):
    qr = pl.program_id(0)
    acc = jnp.zeros((37, 37, 56), jnp.float32)
    for di in range(7):
        for dj in range(7):
            k = di * 7 + dj
            blk = a_ref[pl.ds(di + 4 * qr, 37), dj:dj + 37, :]
            cv = c_ref[0, k, :]
            d = blk - cv[None, None, :]
            acc = acc + d * d
    ii = jax.lax.broadcasted_iota(jnp.int32, (37, 37, 56), 0)
    jj = jax.lax.broadcasted_iota(jnp.int32, (37, 37, 56), 1)
    acc = jnp.where((ii == 18) & (jj == 18), -1.0, acc)
    of = ii * 37 + jj
    big = jnp.float32(3.0e38)
    for r in range(m):
        mn = jnp.min(acc, axis=(0, 1))
        hit = acc == mn[None, None, :]
        idxv = jnp.min(jnp.where(hit, of, 1369), axis=(0, 1))
        o_ref[0, r, :] = idxv
        acc = jnp.where(of == idxv[None, None, :], big, acc)


def _block_matching_pl(xb, m):
    xpad = jnp.pad(xb, ((0, 0), (0, 0), (21, 21), (21, 21)),
                   mode='reflect')[0, 0]
    A = jnp.stack([xpad[:, t:t + 221:4] for t in range(43)], axis=1)
    C2 = jnp.stack(
        [xpad[18 + di:18 + di + 221:4, 18 + dj:18 + dj + 221:4]
         for di in range(7) for dj in range(7)], axis=1)
    o = pl.pallas_call(
        lambda a, b, c: _dist_topk_kernel(a, b, c, m=m),
        grid=(56,),
        in_specs=[
            pl.BlockSpec((266, 43, 56), lambda i: (0, 0, 0)),
            pl.BlockSpec((1, 49, 56), lambda i: (i, 0, 0)),
        ],
        out_specs=pl.BlockSpec((1, m, 56), lambda i: (i, 0, 0)),
        out_shape=jax.ShapeDtypeStruct((56, m, 56), jnp.int32),
    )(A, C2)
    qr = jnp.arange(56)[:, None, None]
    qc = jnp.arange(56)[None, None, :]
    flat = (o // 37 + 4 * qr) * 260 + (o % 37 + 4 * qc)
    return jnp.transpose(flat, (0, 2, 1)).reshape(3136, m)


def _gather_kernel(t_ref, i_ref, o_ref, *, m, base):
    def body(k, carry):
        pos = i_ref[0, 0, k] - base
        o_ref[0, pl.ds(k, 1), :] = t_ref[pl.ds(pos, 1), :]
        return carry

    jax.lax.fori_loop(0, m, body, 0)


def _group_patches_pl(table_t, idx2):
    Bn, m = idx2.shape
    half = Bn // 2
    hrows = 145 * 260
    bases = (0, 112 * 260)
    idx3 = idx2.reshape(Bn, 1, m)
    outs = []
    for h in range(2):
        table_h = jax.lax.dynamic_slice(table_t, (bases[h], 0), (hrows, 147))
        outs.append(pl.pallas_call(
            lambda a, b, c, _h=h: _gather_kernel(a, b, c, m=m,
                                                 base=bases[_h]),
            grid=(half,),
            in_specs=[
                pl.BlockSpec((hrows, 147), lambda i: (0, 0)),
                pl.BlockSpec((1, 1, m), lambda i, _h=h: (_h * half + i, 0, 0),
                             memory_space=pltpu.SMEM),
            ],
            out_specs=pl.BlockSpec((1, m, 147), lambda i: (i, 0, 0)),
            out_shape=jax.ShapeDtypeStruct((half, m, 147), jnp.float32),
        )(table_h, idx3))
    return jnp.concatenate(outs, axis=0).reshape(1, Bn, m, 147)


def _scatter_kernel(p_ref, i_ref, o_ref, *, m, base):
    g = pl.program_id(0)

    @pl.when(g == 0)
    def _():
        o_ref[...] = jnp.zeros_like(o_ref)

    def body(k, carry):
        pos = i_ref[0, 0, k] - base
        row = p_ref[0, pl.ds(k, 1), :]
        o_ref[pl.ds(pos, 1), :] = o_ref[pl.ds(pos, 1), :] + row
        return carry

    jax.lax.fori_loop(0, m, body, 0)


def _aggregation_pl(X_hat, weights, idx2, input_y, p):
    N, C, H, Wd = input_y.shape
    w = _WIN
    Bn, m = idx2.shape
    L = (H + w - 1) * (Wd + w - 1)
    P = jnp.concatenate([X_hat * weights, weights], axis=3).reshape(Bn, m, 148)
    idx3 = idx2.reshape(Bn, 1, m)
    half = Bn // 2
    hrows = 145 * 260
    bases = (0, 112 * 260)
    halves = []
    for h in range(2):
        halves.append(pl.pallas_call(
            lambda a, b, c, _h=h: _scatter_kernel(a, b, c, m=m,
                                                  base=bases[_h]),
            grid=(half,),
            in_specs=[
                pl.BlockSpec((1, m, 148), lambda i, _h=h: (_h * half + i, 0, 0)),
                pl.BlockSpec((1, 1, m), lambda i, _h=h: (_h * half + i, 0, 0),
                             memory_space=pltpu.SMEM),
            ],
            out_specs=pl.BlockSpec((hrows, 148), lambda i: (0, 0)),
            out_shape=jax.ShapeDtypeStruct((hrows, 148), jnp.float32),
        )(P, idx3))
    acc = (jnp.zeros((L, 148), jnp.float32)
           .at[:hrows].add(halves[0])
           .at[bases[1]:bases[1] + hrows].add(halves[1]))
    X_sum = acc[:, :147].T.reshape(1, 147, L)
    dsum = acc[:, 147]
    OH = H + w - 1 + p - 1
    OW = Wd + w - 1 + p - 1
    xx = _fold(X_sum, (OH, OW), p)
    dv = _fold(jnp.broadcast_to(dsum[None, None, :], (1, p * p, L)),
               (OH, OW), p)
    den = xx / dv
    r = w // 2 + p // 2
    return den[:, :, r:-r, r:-r]


def _gram_kernel(c_ref, b_ref, g_ref, *, add_ridge):
    B = b_ref[0]
    G = jax.lax.dot_general(B, B, (((1,), (1,)), ((), ())),
                            preferred_element_type=jnp.float32)
    if add_ridge:
        m = G.shape[0]
        eye = (jax.lax.broadcasted_iota(jnp.int32, (m, m), 0) ==
               jax.lax.broadcasted_iota(jnp.int32, (m, m), 1))
        G = G + jnp.where(eye, c_ref[0, 0], 0.0)
    g_ref[0] = G


def _sweep_theta_kernel(c_ref, g_ref, theta_ref, w_ref, *, m):
    M = g_ref[...]
    rows_i = jax.lax.broadcasted_iota(jnp.int32, (1, m, m), 1)
    cols_i = jax.lax.broadcasted_iota(jnp.int32, (1, m, m), 2)
    shape = M.shape
    for j in range(m):
        p = M[:, j:j + 1, j:j + 1]
        col = M[:, :, j:j + 1]
        row = M[:, j:j + 1, :]
        inv_p = 1.0 / p
        M = M - col * row * inv_p
        M = jnp.where(rows_i == j, jnp.broadcast_to(row * inv_p, shape), M)
        M = jnp.where(cols_i == j, jnp.broadcast_to(col * inv_p, shape), M)
        M = jnp.where((rows_i == j) & (cols_i == j),
                      jnp.broadcast_to(-inv_p, shape), M)
    c = c_ref[0, 0]
    theta = jnp.where(rows_i == cols_i, 1.0, 0.0) + c * M
    theta_ref[...] = theta
    w_ref[...] = 1.0 / jnp.sum(theta * theta, axis=2)


def _xhat_kernel(t_ref, y_ref, o_ref):
    o_ref[0] = jax.lax.dot_general(t_ref[0], y_ref[0], (((1,), (0,)), ((), ())),
                                   preferred_element_type=jnp.float32)


def _denoise(Y, basis, sigma, add_ridge):
    _, Bn, m, n = Y.shape
    Yr = Y.reshape(Bn, m, n)
    Br = basis.reshape(Bn, m, n)
    c = (n * sigma ** 2).reshape(1, 1).astype(jnp.float32)

    gram = pl.pallas_call(
        lambda c_ref, b_ref, g_ref: _gram_kernel(c_ref, b_ref, g_ref,
                                                 add_ridge=add_ridge),
        grid=(Bn,),
        in_specs=[
            pl.BlockSpec((1, 1), lambda i: (0, 0)),
            pl.BlockSpec((1, m, n), lambda i: (i, 0, 0)),
        ],
        out_specs=pl.BlockSpec((1, m, m), lambda i: (i, 0, 0)),
        out_shape=jax.ShapeDtypeStruct((Bn, m, m), jnp.float32),
    )(c, Br)

    blk = 64
    nblk = Bn // blk
    theta, w = pl.pallas_call(
        lambda c_ref, g_ref, t_ref, w_ref: _sweep_theta_kernel(
            c_ref, g_ref, t_ref, w_ref, m=m),
        grid=(nblk,),
        in_specs=[
            pl.BlockSpec((1, 1), lambda i: (0, 0)),
            pl.BlockSpec((blk, m, m), lambda i: (i, 0, 0)),
        ],
        out_specs=[
            pl.BlockSpec((blk, m, m), lambda i: (i, 0, 0)),
            pl.BlockSpec((blk, m), lambda i: (i, 0)),
        ],
        out_shape=[
            jax.ShapeDtypeStruct((Bn, m, m), jnp.float32),
            jax.ShapeDtypeStruct((Bn, m), jnp.float32),
        ],
    )(c, gram)

    xhat = pl.pallas_call(
        _xhat_kernel,
        grid=(Bn,),
        in_specs=[
            pl.BlockSpec((1, m, m), lambda i: (i, 0, 0)),
            pl.BlockSpec((1, m, n), lambda i: (i, 0, 0)),
        ],
        out_specs=pl.BlockSpec((1, m, n), lambda i: (i, 0, 0)),
        out_shape=jax.ShapeDtypeStruct((Bn, m, n), jnp.float32),
    )(theta, Yr)

    return xhat.reshape(1, Bn, m, n), w.reshape(1, Bn, m, 1)


def _step1(input_y, sigma):
    N, C, H, Wd = input_y.shape
    p, m, w = _P1, _K1, _WIN
    y_block = jnp.mean(input_y, axis=1, keepdims=True)
    idx2 = _block_matching_pl(y_block, m)
    unfold_y = _reflect_unfold(input_y, p // 2 + w // 2, p)
    Y = _group_patches_pl(unfold_y[0].T, idx2)
    X_hat, weights = _denoise(Y, Y, sigma, add_ridge=False)
    return _aggregation_pl(X_hat, weights, idx2, input_y, p)


def _step2(input_y, input_x, sigma):
    N, C, H, Wd = input_y.shape
    p, m, w = _P2, _K2, _WIN
    x_block = jnp.mean(input_x, axis=1, keepdims=True)
    idx2 = _block_matching_pl(x_block, m)
    unfold_y = _reflect_unfold(input_y, p // 2 + w // 2, p)
    unfold_x = _reflect_unfold(input_x, p // 2 + w // 2, p)
    Y = _group_patches_pl(unfold_y[0].T, idx2)
    X = _group_patches_pl(unfold_x[0].T, idx2)
    X_hat, weights = _denoise(Y, X, sigma, add_ridge=True)
    return _aggregation_pl(X_hat, weights, idx2, input_y, p)


def kernel(input_y, sigma):
    den1 = _step1(input_y, sigma)
    den2 = _step2(input_y, den1, sigma)
    return den2
